# baseline (speedup 1.0000x reference)
import functools

import jax
import jax.numpy as jnp
from jax import lax
from jax.experimental import pallas as pl
from jax.experimental.pallas import tpu as pltpu
from jax.experimental.pallas import tpu_sc as plsc

T, N, E, D, H = 8, 10000, 320000, 128, 64
NPAD = 10240
RPT = 2560
RREAL = E // 128
RPADE = RPT - RREAL
CHT = RPT // 16
NB, BN = 5, 2000
NBL, BNL = 10, 1000

def _sc_mesh():
    return plsc.VectorSubcoreMesh(
        core_axis_name="c", subcore_axis_name="s", num_cores=2, num_subcores=16)


def _tc_edge_prep(ei):
    RB = 320
    NRB = RPT // RB
    EB = RB * 128

    def body(ei_ref, gs_ref, gd_ref):
        t = pl.program_id(0)
        e = pl.program_id(1)
        toff = (t % 4) * N
        src = ei_ref[0, 0]
        dst = ei_ref[0, 1]

        @pl.when(e < NRB - 1)
        def _():
            gs_ref[0] = (src + toff).reshape(RB, 128)
            gd_ref[0] = dst.reshape(RB, 128)

        @pl.when(e == NRB - 1)
        def _():
            j = lax.iota(jnp.int32, EB)
            is_pad = (j + e * EB) >= E
            lane = j & 127
            row8 = (j >> 7) & 7
            ps = lane + row8 * 128
            pd = N + (lane & 15) + row8 * 16
            gs_ref[0] = (jnp.where(is_pad, ps, src) + toff).reshape(RB, 128)
            gd_ref[0] = jnp.where(is_pad, pd, dst).reshape(RB, 128)

    return pl.pallas_call(
        body,
        grid=(T, NRB),
        in_specs=[pl.BlockSpec((1, 2, EB), lambda t, e: (t, 0, e))],
        out_specs=[
            pl.BlockSpec((1, RB, 128), lambda t, e: (t, e, 0)),
            pl.BlockSpec((1, RB, 128), lambda t, e: (t, e, 0)),
        ],
        out_shape=[jax.ShapeDtypeStruct((T, RPT, 128), jnp.int32)] * 2,
    )(ei)


def _tc_mm1(xs, deg, W1, toff):
    def body(x_ref, deg_ref, w_ref, z_ref, dinv_ref):
        dinv = lax.rsqrt(deg_ref[0] + 1.0)
        y = jnp.dot(x_ref[0], w_ref[...], preferred_element_type=jnp.float32)
        z_ref[0] = y * dinv
        dinv_ref[0] = dinv

    return pl.pallas_call(
        body,
        grid=(4, NB),
        in_specs=[
            pl.BlockSpec((1, BN, D), lambda t, n: (t + toff, n, 0)),
            pl.BlockSpec((1, BN, 1), lambda t, n: (t + toff, n, 0)),
            pl.BlockSpec((D, H), lambda t, n: (0, 0)),
        ],
        out_specs=[
            pl.BlockSpec((1, BN, H), lambda t, n: (t, n, 0)),
            pl.BlockSpec((1, BN, 1), lambda t, n: (t, n, 0)),
        ],
        out_shape=[
            jax.ShapeDtypeStruct((4, N, H), jnp.float32),
            jax.ShapeDtypeStruct((4, N, 1), jnp.float32),
        ],
    )(xs, deg, W1)


def _tc_mm2(o1, z1, dinv, W2, b1r):
    def body(o_ref, z_ref, dinv_ref, w_ref, b_ref, z2_ref):
        dinv = dinv_ref[0]
        h = jnp.maximum((o_ref[0] + z_ref[0]) * dinv + b_ref[...], 0.0)
        z2_ref[0] = jnp.dot(
            h, w_ref[...], preferred_element_type=jnp.float32) * dinv

    return pl.pallas_call(
        body,
        grid=(4, NB),
        in_specs=[
            pl.BlockSpec((1, BN, H), lambda t, n: (t, n, 0)),
            pl.BlockSpec((1, BN, H), lambda t, n: (t, n, 0)),
            pl.BlockSpec((1, BN, 1), lambda t, n: (t, n, 0)),
            pl.BlockSpec((H, H), lambda t, n: (0, 0)),
            pl.BlockSpec((1, H), lambda t, n: (0, 0)),
        ],
        out_specs=[pl.BlockSpec((1, BN, H), lambda t, n: (t, n, 0))],
        out_shape=[jax.ShapeDtypeStruct((4, N, H), jnp.float32)],
    )(o1, z1, dinv, W2, b1r)[0]


def _tc_lstm_q(o2, z2, dinv, toff, b2r, wih_s, whh_s, bsum, carry, head_w):
    first = carry is None
    head = head_w is not None

    def body(*refs):
        i = 0
        o_ref, z_ref, dinv_ref, b_ref, wih_ref, whh_ref, bs_ref = refs[:7]
        i = 7
        if not first:
            hin_ref, cin_ref = refs[i:i + 2]
            i += 2
        if head:
            wh_ref, bh_ref = refs[i:i + 2]
            i += 2
        h_out, c_out = refs[i:i + 2]
        i += 2
        if head:
            out_ref = refs[i]
            i += 1
        h_s, c_s = refs[i:i + 2]
        t = pl.program_id(1)

        @pl.when(t == 0)
        def _():
            if first:
                h_s[...] = jnp.zeros_like(h_s)
                c_s[...] = jnp.zeros_like(c_s)
            else:
                h_s[...] = hin_ref[...]
                c_s[...] = cin_ref[...]

        dinv = dinv_ref[0]
        seq = jnp.maximum((o_ref[0] + z_ref[0]) * dinv + b_ref[...], 0.0)
        hp = h_s[...]
        gates = [
            jnp.dot(seq, wih_ref[k], preferred_element_type=jnp.float32)
            + jnp.dot(hp, whh_ref[k], preferred_element_type=jnp.float32)
            + bs_ref[k]
            for k in range(4)
        ]
        gi = jax.nn.sigmoid(gates[0])
        gf = jax.nn.sigmoid(gates[1])
        gg = jnp.tanh(gates[2])
        go = jax.nn.sigmoid(gates[3])
        c = gf * c_s[...] + gi * gg
        h = go * jnp.tanh(c)
        c_s[...] = c
        h_s[...] = h

        @pl.when(t == 1)
        def _():
            h_out[...] = h
            c_out[...] = c
            if head:
                out_ref[...] = (jnp.sum(h * wh_ref[...], axis=1,
                                        keepdims=True) + bh_ref[...])

    _m3 = lambda n, t: (t, n, 0)
    _m3o = lambda n, t: (t + toff, n, 0)
    _m2 = lambda n, t: (n, 0)
    _m0 = lambda n, t: (0, 0)
    in_specs = [
        pl.BlockSpec((1, BNL, H), _m3),
        pl.BlockSpec((1, BNL, H), _m3o),
        pl.BlockSpec((1, BNL, 1), _m3o),
        pl.BlockSpec((1, H), _m0),
        pl.BlockSpec((4, H, H), lambda n, t: (0, 0, 0)),
        pl.BlockSpec((4, H, H), lambda n, t: (0, 0, 0)),
        pl.BlockSpec((4, 1, H), lambda n, t: (0, 0, 0)),
    ]
    args = [o2, z2, dinv, b2r, wih_s, whh_s, bsum]
    if not first:
        in_specs += [pl.BlockSpec((BNL, H), _m2), pl.BlockSpec((BNL, H), _m2)]
        args += list(carry)
    if head:
        in_specs += [pl.BlockSpec((1, H), _m0), pl.BlockSpec((1, 1), _m0)]
        args += list(head_w)
    out_specs = [pl.BlockSpec((BNL, H), _m2), pl.BlockSpec((BNL, H), _m2)]
    out_shape = [jax.ShapeDtypeStruct((N, H), jnp.float32)] * 2
    if head:
        out_specs += [pl.BlockSpec((BNL, 1), _m2)]
        out_shape += [jax.ShapeDtypeStruct((N, 1), jnp.float32)]
    return pl.pallas_call(
        body,
        grid=(NBL, 2),
        in_specs=in_specs,
        out_specs=out_specs,
        out_shape=out_shape,
        scratch_shapes=[
            pltpu.VMEM((BNL, H), jnp.float32),
            pltpu.VMEM((BNL, H), jnp.float32),
        ],
    )(*args)


def _sc_deg(gdst2d):
    @functools.partial(
        pl.kernel,
        out_type=jax.ShapeDtypeStruct((T * NPAD,), jnp.float32),
        mesh=_sc_mesh(),
        scratch_types=[
            pltpu.VMEM((CHT, 128), jnp.int32),
            pltpu.VMEM((128,), jnp.float32),
            pltpu.VMEM((640,), jnp.float32),
            pltpu.VMEM_SHARED((NPAD,), jnp.float32),
            pltpu.SemaphoreType.DMA,
        ],
        compiler_params=pltpu.CompilerParams(use_tc_tiling_on_sc=False),
    )
    def k(gd_hbm, deg_hbm, didx, ones, zbuf, dacc, ssem):
        c = lax.axis_index("c")
        s = lax.axis_index("s")
        for i in range(8):
            ones[pl.ds(i * 16, 16)] = jnp.ones((16,), jnp.float32)

        def zb(i, carry):
            zbuf[pl.ds(i * 16, 16)] = jnp.zeros((16,), jnp.float32)
            return carry

        lax.fori_loop(0, 40, zb, 0)

        PD = 8
        for j in range(4):
            t = c * 4 + j
            pltpu.sync_copy(zbuf, dacc.at[pl.ds(s * 640, 640)])
            plsc.subcore_barrier()
            pltpu.sync_copy(gd_hbm.at[t, pl.ds(s * CHT, CHT)], didx)
            for m in range(PD):
                pltpu.async_copy(ones, dacc.at[didx.at[m]], ssem, add=True)

            def body(m, carry):
                pltpu.make_async_copy(ones, dacc.at[didx.at[0]], ssem).wait()
                pltpu.async_copy(ones, dacc.at[didx.at[m]], ssem, add=True)
                return carry

            lax.fori_loop(PD, CHT, body, 0)
            for _ in range(PD):
                pltpu.make_async_copy(ones, dacc.at[didx.at[0]], ssem).wait()
            plsc.subcore_barrier()
            pltpu.sync_copy(dacc.at[pl.ds(s * 640, 640)],
                            deg_hbm.at[pl.ds(t * NPAD + s * 640, 640)])
            plsc.subcore_barrier()

    return k(gdst2d)


def _sc_conv(z2d, gsrc2d, gdst2d, tbase, npc):
    @functools.partial(
        pl.kernel,
        out_type=jax.ShapeDtypeStruct((2 * npc * N, H), jnp.float32),
        mesh=_sc_mesh(),
        scratch_types=[
            pltpu.VMEM((CHT, 128), jnp.int32),
            pltpu.VMEM((CHT, 128), jnp.int32),
            pltpu.VMEM((128, H), jnp.float32),
            pltpu.VMEM((128, H), jnp.float32),
            pltpu.VMEM((128, H), jnp.float32),
            pltpu.VMEM((128, H), jnp.float32),
            pltpu.VMEM((128, H), jnp.float32),
            pltpu.VMEM_SHARED((NPAD, H), jnp.float32),
            pltpu.SemaphoreType.DMA,
            pltpu.SemaphoreType.DMA,
            pltpu.SemaphoreType.DMA,
            pltpu.SemaphoreType.DMA,
            pltpu.SemaphoreType.DMA,
            pltpu.SemaphoreType.DMA,
            pltpu.SemaphoreType.DMA,
            pltpu.SemaphoreType.DMA,
        ],
        compiler_params=pltpu.CompilerParams(use_tc_tiling_on_sc=False),
    )
    def k(z_hbm, gs_hbm, gd_hbm, out_hbm, sidx, didx, rb0, rb1, rb2, rb3,
          zbuf, acc, g0, g1, g2, g3, s0, s1, s2, s3):
        c = lax.axis_index("c")
        s = lax.axis_index("s")
        rbs = (rb0, rb1, rb2, rb3)
        gsems = (g0, g1, g2, g3)
        ssems = (s0, s1, s2, s3)

        def zb(i, carry):
            for jj in range(4):
                zbuf[i, pl.ds(jj * 16, 16)] = jnp.zeros((16,), jnp.float32)
            return carry

        lax.fori_loop(0, 128, zb, 0)

        def zero_own():
            for q in range(4):
                pltpu.sync_copy(zbuf, acc.at[pl.ds(s * 624 + q * 128, 128)])
            pltpu.sync_copy(zbuf.at[pl.ds(0, 112)],
                            acc.at[pl.ds(s * 624 + 512, 112)])

            @pl.when(s == 15)
            def _():
                pltpu.sync_copy(zbuf, acc.at[pl.ds(9984, 128)])
                pltpu.sync_copy(zbuf, acc.at[pl.ds(10112, 128)])

        zero_own()

        for j in range(npc):
            tloc = c * npc + j
            t = tbase + tloc
            plsc.subcore_barrier()
            pltpu.sync_copy(gs_hbm.at[t, pl.ds(s * CHT, CHT)], sidx)
            pltpu.sync_copy(gd_hbm.at[t, pl.ds(s * CHT, CHT)], didx)
            for b in range(4):
                pltpu.async_copy(z_hbm.at[sidx.at[b]], rbs[b], gsems[b])

            def body(g, carry):
                for b in range(4):
                    m = 4 * g + b
                    pltpu.make_async_copy(
                        z_hbm.at[sidx.at[m]], rbs[b], gsems[b]).wait()
                    pltpu.async_copy(
                        rbs[b], acc.at[didx.at[m]], ssems[b], add=True)
                for b in range(4):
                    m2 = 4 * (g + 1) + b
                    pltpu.make_async_copy(
                        rbs[b], acc.at[didx.at[0]], ssems[b]).wait()
                    pltpu.async_copy(z_hbm.at[sidx.at[m2]], rbs[b], gsems[b])
                return carry

            lax.fori_loop(0, CHT // 4 - 1, body, 0)
            for b in range(4):
                m = CHT - 4 + b
                pltpu.make_async_copy(
                    z_hbm.at[sidx.at[m]], rbs[b], gsems[b]).wait()
                pltpu.async_copy(rbs[b], acc.at[didx.at[m]], ssems[b], add=True)
            for b in range(4):
                pltpu.make_async_copy(rbs[b], acc.at[didx.at[0]], ssems[b]).wait()
            plsc.subcore_barrier()
            pltpu.sync_copy(acc.at[pl.ds(s * 624, 624)],
                            out_hbm.at[pl.ds(tloc * N + s * 624, 624)])

            @pl.when(s == 15)
            def _():
                pltpu.sync_copy(acc.at[pl.ds(9984, 16)],
                                out_hbm.at[pl.ds(tloc * N + 9984, 16)])

            zero_own()

    return k(z2d, gsrc2d, gdst2d)


def kernel(xs_list, edge_index_list, W1, b1, W2, b2, Wih, Whh, bih, bhh, Wh, bh):
    gsrc2d, gdst2d = _tc_edge_prep(edge_index_list)

    deg = _sc_deg(gdst2d).reshape(T, NPAD, 1)
    b1r = b1.reshape(1, H)
    z1a, dinva = _tc_mm1(xs_list, deg, W1, 0)
    z1b, dinvb = _tc_mm1(xs_list, deg, W1, 4)
    o1a = _sc_conv(z1a.reshape(4 * N, H), gsrc2d, gdst2d, 0, 2)
    o1b = _sc_conv(z1b.reshape(4 * N, H), gsrc2d, gdst2d, 4, 2)
    z2a = _tc_mm2(o1a.reshape(4, N, H), z1a, dinva, W2, b1r)
    z2b = _tc_mm2(o1b.reshape(4, N, H), z1b, dinvb, W2, b1r)
    wih_s = jnp.stack([Wih[k * H:(k + 1) * H, :].T for k in range(4)])
    whh_s = jnp.stack([Whh[k * H:(k + 1) * H, :].T for k in range(4)])
    bsum = (bih + bhh).reshape(4, 1, H)
    b2r = b2.reshape(1, H)
    z2aF = z2a.reshape(4 * N, H)
    z2bF = z2b.reshape(4 * N, H)
    lw = (b2r, wih_s, whh_s, bsum)
    o2q0 = _sc_conv(z2aF, gsrc2d, gdst2d, 0, 1)
    h0, c0 = _tc_lstm_q(o2q0.reshape(2, N, H), z2a, dinva, 0, *lw,
                        None, None)
    o2q1 = _sc_conv(z2aF, gsrc2d, gdst2d, 2, 1)
    z2bF1, h0b, c0b = lax.optimization_barrier((z2bF, h0, c0))
    h1, c1 = _tc_lstm_q(o2q1.reshape(2, N, H), z2a, dinva, 2, *lw,
                        (h0b, c0b), None)
    o2q2 = _sc_conv(z2bF1, gsrc2d, gdst2d, 4, 1)
    z2bF2, h1b, c1b = lax.optimization_barrier((z2bF1, h1, c1))
    h2, c2 = _tc_lstm_q(o2q2.reshape(2, N, H), z2b, dinvb, 0, *lw,
                        (h1b, c1b), None)
    o2q3 = _sc_conv(z2bF2, gsrc2d, gdst2d, 6, 1)
    _, _, out2d = _tc_lstm_q(o2q3.reshape(2, N, H), z2b, dinvb, 2, *lw,
                             (h2, c2), (Wh.reshape(1, H), bh.reshape(1, 1)))
    return out2d.reshape(-1)


# speedup vs baseline: 1.0086x; 1.0086x over previous
import functools

import jax
import jax.numpy as jnp
from jax import lax
from jax.experimental import pallas as pl
from jax.experimental.pallas import tpu as pltpu
from jax.experimental.pallas import tpu_sc as plsc

T, N, E, D, H = 8, 10000, 320000, 128, 64
NPAD = 10240
RPT = 2560
RREAL = E // 128
RPADE = RPT - RREAL
CHT = RPT // 16
NB, BN = 5, 2000
NBL, BNL = 10, 1000

def _sc_mesh():
    return plsc.VectorSubcoreMesh(
        core_axis_name="c", subcore_axis_name="s", num_cores=2, num_subcores=16)


def _tc_edge_prep(ei):
    RB = 320
    NRB = RPT // RB
    EB = RB * 128

    def body(ei_ref, gs_ref, gd_ref):
        t = pl.program_id(0)
        e = pl.program_id(1)
        toff = (t % 4) * N
        src = ei_ref[0, 0]
        dst = ei_ref[0, 1]

        @pl.when(e < NRB - 1)
        def _():
            gs_ref[0] = (src + toff).reshape(RB, 128)
            gd_ref[0] = dst.reshape(RB, 128)

        @pl.when(e == NRB - 1)
        def _():
            j = lax.iota(jnp.int32, EB)
            is_pad = (j + e * EB) >= E
            lane = j & 127
            row8 = (j >> 7) & 7
            ps = lane + row8 * 128
            pd = N + (lane & 15) + row8 * 16
            gs_ref[0] = (jnp.where(is_pad, ps, src) + toff).reshape(RB, 128)
            gd_ref[0] = jnp.where(is_pad, pd, dst).reshape(RB, 128)

    return pl.pallas_call(
        body,
        grid=(T, NRB),
        in_specs=[pl.BlockSpec((1, 2, EB), lambda t, e: (t, 0, e))],
        out_specs=[
            pl.BlockSpec((1, RB, 128), lambda t, e: (t, e, 0)),
            pl.BlockSpec((1, RB, 128), lambda t, e: (t, e, 0)),
        ],
        out_shape=[jax.ShapeDtypeStruct((T, RPT, 128), jnp.int32)] * 2,
    )(ei)


def _tc_mm1(xs, deg, W1, toff):
    def body(x_ref, deg_ref, w_ref, z_ref, dinv_ref):
        dinv = lax.rsqrt(deg_ref[0] + 1.0)
        y = jnp.dot(x_ref[0], w_ref[...], preferred_element_type=jnp.float32)
        z_ref[0] = y * dinv
        dinv_ref[0] = dinv

    return pl.pallas_call(
        body,
        grid=(4, NB),
        in_specs=[
            pl.BlockSpec((1, BN, D), lambda t, n: (t + toff, n, 0)),
            pl.BlockSpec((1, BN, 1), lambda t, n: (t + toff, n, 0)),
            pl.BlockSpec((D, H), lambda t, n: (0, 0)),
        ],
        out_specs=[
            pl.BlockSpec((1, BN, H), lambda t, n: (t, n, 0)),
            pl.BlockSpec((1, BN, 1), lambda t, n: (t, n, 0)),
        ],
        out_shape=[
            jax.ShapeDtypeStruct((4, N, H), jnp.float32),
            jax.ShapeDtypeStruct((4, N, 1), jnp.float32),
        ],
    )(xs, deg, W1)


def _tc_mm2(o1, z1, dinv, W2, b1r):
    def body(o_ref, z_ref, dinv_ref, w_ref, b_ref, z2_ref):
        dinv = dinv_ref[0]
        h = jnp.maximum((o_ref[0] + z_ref[0]) * dinv + b_ref[...], 0.0)
        z2_ref[0] = jnp.dot(
            h, w_ref[...], preferred_element_type=jnp.float32) * dinv

    return pl.pallas_call(
        body,
        grid=(4, NB),
        in_specs=[
            pl.BlockSpec((1, BN, H), lambda t, n: (t, n, 0)),
            pl.BlockSpec((1, BN, H), lambda t, n: (t, n, 0)),
            pl.BlockSpec((1, BN, 1), lambda t, n: (t, n, 0)),
            pl.BlockSpec((H, H), lambda t, n: (0, 0)),
            pl.BlockSpec((1, H), lambda t, n: (0, 0)),
        ],
        out_specs=[pl.BlockSpec((1, BN, H), lambda t, n: (t, n, 0))],
        out_shape=[jax.ShapeDtypeStruct((4, N, H), jnp.float32)],
    )(o1, z1, dinv, W2, b1r)[0]


def _tc_lstm_half(o2, z2, dinv, b2r, wih_s, whh_s, bsum, carry, head_w):
    first = carry is None
    head = head_w is not None

    def body(*refs):
        i = 0
        o_ref, z_ref, dinv_ref, b_ref, wih_ref, whh_ref, bs_ref = refs[:7]
        i = 7
        if not first:
            hin_ref, cin_ref = refs[i:i + 2]
            i += 2
        if head:
            wh_ref, bh_ref = refs[i:i + 2]
            i += 2
        h_out, c_out = refs[i:i + 2]
        i += 2
        if head:
            out_ref = refs[i]
            i += 1
        h_s, c_s = refs[i:i + 2]
        t = pl.program_id(1)

        @pl.when(t == 0)
        def _():
            if first:
                h_s[...] = jnp.zeros_like(h_s)
                c_s[...] = jnp.zeros_like(c_s)
            else:
                h_s[...] = hin_ref[...]
                c_s[...] = cin_ref[...]

        dinv = dinv_ref[0]
        seq = jnp.maximum((o_ref[0] + z_ref[0]) * dinv + b_ref[...], 0.0)
        hp = h_s[...]
        gates = [
            jnp.dot(seq, wih_ref[k], preferred_element_type=jnp.float32)
            + jnp.dot(hp, whh_ref[k], preferred_element_type=jnp.float32)
            + bs_ref[k]
            for k in range(4)
        ]
        gi = jax.nn.sigmoid(gates[0])
        gf = jax.nn.sigmoid(gates[1])
        gg = jnp.tanh(gates[2])
        go = jax.nn.sigmoid(gates[3])
        c = gf * c_s[...] + gi * gg
        h = go * jnp.tanh(c)
        c_s[...] = c
        h_s[...] = h

        @pl.when(t == 3)
        def _():
            h_out[...] = h
            c_out[...] = c
            if head:
                out_ref[...] = (jnp.sum(h * wh_ref[...], axis=1,
                                        keepdims=True) + bh_ref[...])

    _m3 = lambda n, t: (t, n, 0)
    _m2 = lambda n, t: (n, 0)
    _m0 = lambda n, t: (0, 0)
    in_specs = [
        pl.BlockSpec((1, BNL, H), _m3),
        pl.BlockSpec((1, BNL, H), _m3),
        pl.BlockSpec((1, BNL, 1), _m3),
        pl.BlockSpec((1, H), _m0),
        pl.BlockSpec((4, H, H), lambda n, t: (0, 0, 0)),
        pl.BlockSpec((4, H, H), lambda n, t: (0, 0, 0)),
        pl.BlockSpec((4, 1, H), lambda n, t: (0, 0, 0)),
    ]
    args = [o2, z2, dinv, b2r, wih_s, whh_s, bsum]
    if not first:
        in_specs += [pl.BlockSpec((BNL, H), _m2), pl.BlockSpec((BNL, H), _m2)]
        args += list(carry)
    if head:
        in_specs += [pl.BlockSpec((1, H), _m0), pl.BlockSpec((1, 1), _m0)]
        args += list(head_w)
    out_specs = [pl.BlockSpec((BNL, H), _m2), pl.BlockSpec((BNL, H), _m2)]
    out_shape = [jax.ShapeDtypeStruct((N, H), jnp.float32)] * 2
    if head:
        out_specs += [pl.BlockSpec((BNL, 1), _m2)]
        out_shape += [jax.ShapeDtypeStruct((N, 1), jnp.float32)]
    return pl.pallas_call(
        body,
        grid=(NBL, 4),
        in_specs=in_specs,
        out_specs=out_specs,
        out_shape=out_shape,
        scratch_shapes=[
            pltpu.VMEM((BNL, H), jnp.float32),
            pltpu.VMEM((BNL, H), jnp.float32),
        ],
    )(*args)


def _sc_deg(gdst2d):
    @functools.partial(
        pl.kernel,
        out_type=jax.ShapeDtypeStruct((T * NPAD,), jnp.float32),
        mesh=_sc_mesh(),
        scratch_types=[
            pltpu.VMEM((CHT, 128), jnp.int32),
            pltpu.VMEM((128,), jnp.float32),
            pltpu.VMEM((640,), jnp.float32),
            pltpu.VMEM_SHARED((NPAD,), jnp.float32),
            pltpu.SemaphoreType.DMA,
        ],
        compiler_params=pltpu.CompilerParams(use_tc_tiling_on_sc=False),
    )
    def k(gd_hbm, deg_hbm, didx, ones, zbuf, dacc, ssem):
        c = lax.axis_index("c")
        s = lax.axis_index("s")
        for i in range(8):
            ones[pl.ds(i * 16, 16)] = jnp.ones((16,), jnp.float32)

        def zb(i, carry):
            zbuf[pl.ds(i * 16, 16)] = jnp.zeros((16,), jnp.float32)
            return carry

        lax.fori_loop(0, 40, zb, 0)

        PD = 8
        for j in range(4):
            t = c * 4 + j
            pltpu.sync_copy(zbuf, dacc.at[pl.ds(s * 640, 640)])
            plsc.subcore_barrier()
            pltpu.sync_copy(gd_hbm.at[t, pl.ds(s * CHT, CHT)], didx)
            for m in range(PD):
                pltpu.async_copy(ones, dacc.at[didx.at[m]], ssem, add=True)

            def body(m, carry):
                pltpu.make_async_copy(ones, dacc.at[didx.at[0]], ssem).wait()
                pltpu.async_copy(ones, dacc.at[didx.at[m]], ssem, add=True)
                return carry

            lax.fori_loop(PD, CHT, body, 0)
            for _ in range(PD):
                pltpu.make_async_copy(ones, dacc.at[didx.at[0]], ssem).wait()
            plsc.subcore_barrier()
            pltpu.sync_copy(dacc.at[pl.ds(s * 640, 640)],
                            deg_hbm.at[pl.ds(t * NPAD + s * 640, 640)])
            plsc.subcore_barrier()

    return k(gdst2d)


def _sc_conv(z2d, gsrc2d, gdst2d, tbase):
    @functools.partial(
        pl.kernel,
        out_type=jax.ShapeDtypeStruct((4 * N, H), jnp.float32),
        mesh=_sc_mesh(),
        scratch_types=[
            pltpu.VMEM((CHT, 128), jnp.int32),
            pltpu.VMEM((CHT, 128), jnp.int32),
            pltpu.VMEM((128, H), jnp.float32),
            pltpu.VMEM((128, H), jnp.float32),
            pltpu.VMEM((128, H), jnp.float32),
            pltpu.VMEM((128, H), jnp.float32),
            pltpu.VMEM((128, H), jnp.float32),
            pltpu.VMEM_SHARED((NPAD, H), jnp.float32),
            pltpu.SemaphoreType.DMA,
            pltpu.SemaphoreType.DMA,
            pltpu.SemaphoreType.DMA,
            pltpu.SemaphoreType.DMA,
            pltpu.SemaphoreType.DMA,
            pltpu.SemaphoreType.DMA,
            pltpu.SemaphoreType.DMA,
            pltpu.SemaphoreType.DMA,
        ],
        compiler_params=pltpu.CompilerParams(use_tc_tiling_on_sc=False),
    )
    def k(z_hbm, gs_hbm, gd_hbm, out_hbm, sidx, didx, rb0, rb1, rb2, rb3,
          zbuf, acc, g0, g1, g2, g3, s0, s1, s2, s3):
        c = lax.axis_index("c")
        s = lax.axis_index("s")
        rbs = (rb0, rb1, rb2, rb3)
        gsems = (g0, g1, g2, g3)
        ssems = (s0, s1, s2, s3)

        def zb(i, carry):
            for jj in range(4):
                zbuf[i, pl.ds(jj * 16, 16)] = jnp.zeros((16,), jnp.float32)
            return carry

        lax.fori_loop(0, 128, zb, 0)

        def zero_own():
            for q in range(4):
                pltpu.sync_copy(zbuf, acc.at[pl.ds(s * 624 + q * 128, 128)])
            pltpu.sync_copy(zbuf.at[pl.ds(0, 112)],
                            acc.at[pl.ds(s * 624 + 512, 112)])

            @pl.when(s == 15)
            def _():
                pltpu.sync_copy(zbuf, acc.at[pl.ds(9984, 128)])
                pltpu.sync_copy(zbuf, acc.at[pl.ds(10112, 128)])

        zero_own()

        for j in range(2):
            tloc = c * 2 + j
            t = tbase + tloc
            plsc.subcore_barrier()
            pltpu.sync_copy(gs_hbm.at[t, pl.ds(s * CHT, CHT)], sidx)
            pltpu.sync_copy(gd_hbm.at[t, pl.ds(s * CHT, CHT)], didx)
            for b in range(4):
                pltpu.async_copy(z_hbm.at[sidx.at[b]], rbs[b], gsems[b])

            def body(g, carry):
                for b in range(4):
                    m = 4 * g + b
                    pltpu.make_async_copy(
                        z_hbm.at[sidx.at[m]], rbs[b], gsems[b]).wait()
                    pltpu.async_copy(
                        rbs[b], acc.at[didx.at[m]], ssems[b], add=True)
                for b in range(4):
                    m2 = 4 * (g + 1) + b
                    pltpu.make_async_copy(
                        rbs[b], acc.at[didx.at[0]], ssems[b]).wait()
                    pltpu.async_copy(z_hbm.at[sidx.at[m2]], rbs[b], gsems[b])
                return carry

            lax.fori_loop(0, CHT // 4 - 1, body, 0)
            for b in range(4):
                m = CHT - 4 + b
                pltpu.make_async_copy(
                    z_hbm.at[sidx.at[m]], rbs[b], gsems[b]).wait()
                pltpu.async_copy(rbs[b], acc.at[didx.at[m]], ssems[b], add=True)
            for b in range(4):
                pltpu.make_async_copy(rbs[b], acc.at[didx.at[0]], ssems[b]).wait()
            plsc.subcore_barrier()
            pltpu.sync_copy(acc.at[pl.ds(s * 624, 624)],
                            out_hbm.at[pl.ds(tloc * N + s * 624, 624)])

            @pl.when(s == 15)
            def _():
                pltpu.sync_copy(acc.at[pl.ds(9984, 16)],
                                out_hbm.at[pl.ds(tloc * N + 9984, 16)])

            zero_own()

    return k(z2d, gsrc2d, gdst2d)


def kernel(xs_list, edge_index_list, W1, b1, W2, b2, Wih, Whh, bih, bhh, Wh, bh):
    gsrc2d, gdst2d = _tc_edge_prep(edge_index_list)

    deg = _sc_deg(gdst2d).reshape(T, NPAD, 1)
    b1r = b1.reshape(1, H)
    z1a, dinva = _tc_mm1(xs_list, deg, W1, 0)
    z1b, dinvb = _tc_mm1(xs_list, deg, W1, 4)
    o1a = _sc_conv(z1a.reshape(4 * N, H), gsrc2d, gdst2d, 0)
    o1b = _sc_conv(z1b.reshape(4 * N, H), gsrc2d, gdst2d, 4)
    z2a = _tc_mm2(o1a.reshape(4, N, H), z1a, dinva, W2, b1r)
    z2b = _tc_mm2(o1b.reshape(4, N, H), z1b, dinvb, W2, b1r)
    wih_s = jnp.stack([Wih[k * H:(k + 1) * H, :].T for k in range(4)])
    whh_s = jnp.stack([Whh[k * H:(k + 1) * H, :].T for k in range(4)])
    bsum = (bih + bhh).reshape(4, 1, H)
    b2r = b2.reshape(1, H)
    o2a = _sc_conv(z2a.reshape(4 * N, H), gsrc2d, gdst2d, 0)
    h3, c3 = _tc_lstm_half(o2a.reshape(4, N, H), z2a, dinva, b2r, wih_s,
                           whh_s, bsum, None, None)
    o2b = _sc_conv(z2b.reshape(4 * N, H), gsrc2d, gdst2d, 4)
    _, _, out2d = _tc_lstm_half(o2b.reshape(4, N, H), z2b, dinvb, b2r, wih_s,
                                whh_s, bsum, (h3, c3),
                                (Wh.reshape(1, H), bh.reshape(1, 1)))
    return out2d.reshape(-1)
